# baseline (speedup 1.0000x reference)
import jax
import jax.numpy as jnp
from jax.experimental import pallas as pl
from jax.experimental.pallas import tpu as pltpu

_ALPHA_M1 = -0.09999999999999998


def _center_kernel(labels_ref, features_ref, centers_ref, out_ref):
    out_ref[...] = centers_ref[...]
    lane_group = jax.lax.broadcasted_iota(jnp.int32, (1, 128), 1) // 64

    def body(i, carry):
        l = labels_ref[i]
        srow = l // 2
        half = l % 2
        frow = i // 2
        fhalf = i % 2
        cvec = centers_ref[pl.ds(srow, 1), :]
        fvec = features_ref[pl.ds(frow, 1), :]
        fvec_rot = pltpu.roll(fvec, 64, 1)
        f_aligned = jnp.where(half == fhalf, fvec, fvec_rot)
        mask = lane_group == half
        delta = jnp.where(mask, _ALPHA_M1 * (cvec - f_aligned), 0.0)
        out_ref[pl.ds(srow, 1), :] = out_ref[pl.ds(srow, 1), :] + delta
        return carry

    jax.lax.fori_loop(0, labels_ref.shape[0], body, 0)


def kernel(features, labels, centers):
    n, d = centers.shape
    b = features.shape[0]
    out2 = pl.pallas_call(
        _center_kernel,
        out_shape=jax.ShapeDtypeStruct((n // 2, 2 * d), centers.dtype),
        in_specs=[
            pl.BlockSpec(memory_space=pltpu.SMEM),
            pl.BlockSpec(memory_space=pltpu.VMEM),
            pl.BlockSpec(memory_space=pltpu.VMEM),
        ],
        out_specs=pl.BlockSpec(memory_space=pltpu.VMEM),
    )(labels, features.reshape(b // 2, 2 * d), centers.reshape(n // 2, 2 * d))
    return out2.reshape(n, d)


# speedup vs baseline: 6.1453x; 6.1453x over previous
import jax
import jax.numpy as jnp
from jax import lax
from jax.experimental import pallas as pl
from jax.experimental.pallas import tpu as pltpu
from jax.experimental.pallas import tpu_sc as plsc

_AM1 = -0.09999999999999998

_N = 100000
_D = 64
_B = 16384
_NC = 2
_NS = 16
_BT = _B // _NS
_CHUNK = 128
_NCHUNK = _BT // _CHUNK
_NPASS = 4
_SHARD = _N // (_NC * _NPASS)
_NDUMMY = 64
_ROWS_PT = 784


def _sc_body(feat_hbm, lab_hbm, cent_hbm, out_hbm,
             featc_v, delta_v, lab_v, idx_v, table_sh):
    c = lax.axis_index("c")
    s = lax.axis_index("s")
    item0 = s * _BT

    pltpu.sync_copy(lab_hbm.at[pl.ds(s * _NCHUNK, _NCHUNK)], lab_v)

    for j in range(_NCHUNK):
        pltpu.sync_copy(feat_hbm.at[pl.ds(item0 + j * _CHUNK, _CHUNK)], featc_v)
        pltpu.sync_copy(cent_hbm.at[lab_v.at[j]],
                        delta_v.at[pl.ds(j * _CHUNK, _CHUNK)])

        def _delta(i, carry, j=j):
            for d in range(_D // 16):
                sl = pl.ds(d * 16, 16)
                delta_v[j * _CHUNK + i, sl] = _AM1 * (
                    delta_v[j * _CHUNK + i, sl] - featc_v[i, sl])
            return carry

        lax.fori_loop(0, _CHUNK, _delta, 0)

    for p in range(_NPASS):
        shard_base = (2 * p + c) * _SHARD

        rstart = jnp.minimum(s * _ROWS_PT, _SHARD - _ROWS_PT)
        pltpu.sync_copy(cent_hbm.at[pl.ds(shard_base + rstart, _ROWS_PT)],
                        table_sh.at[pl.ds(rstart, _ROWS_PT)])
        plsc.subcore_barrier()

        for k in range(_BT // 16):
            sl = pl.ds((k % 8) * 16, 16)
            lab = lab_v[k // 8, sl]
            rel = lab - shard_base
            in_range = (rel >= 0) & (rel < _SHARD)
            dummy = _SHARD + ((lax.iota(jnp.int32, 16) + k * 16 + s) & (_NDUMMY - 1))
            idx_v[k // 8, sl] = jnp.where(in_range, rel, dummy)

        for j in range(_NCHUNK):
            pltpu.sync_copy(delta_v.at[pl.ds(j * _CHUNK, _CHUNK)],
                            table_sh.at[idx_v.at[j]], add=True)
        plsc.subcore_barrier()

        pltpu.sync_copy(table_sh.at[pl.ds(rstart, _ROWS_PT)],
                        out_hbm.at[pl.ds(shard_base + rstart, _ROWS_PT)])
        plsc.subcore_barrier()


@jax.jit
def _center_update(features, labels2d, centers):
    mesh = plsc.VectorSubcoreMesh(core_axis_name="c", subcore_axis_name="s",
                                  num_cores=_NC, num_subcores=_NS)
    f = pl.kernel(
        _sc_body,
        out_type=jax.ShapeDtypeStruct((_N, _D), jnp.float32),
        mesh=mesh,
        compiler_params=pltpu.CompilerParams(use_tc_tiling_on_sc=False),
        scratch_types=[
            pltpu.VMEM((_CHUNK, _D), jnp.float32),
            pltpu.VMEM((_BT, _D), jnp.float32),
            pltpu.VMEM((_NCHUNK, _CHUNK), jnp.int32),
            pltpu.VMEM((_NCHUNK, _CHUNK), jnp.int32),
            pltpu.VMEM_SHARED((_SHARD + _NDUMMY, _D), jnp.float32),
        ],
    )
    return f(features, labels2d, centers)


def kernel(features, labels, centers):
    labels2d = labels.astype(jnp.int32).reshape(_B // _CHUNK, _CHUNK)
    return _center_update(features, labels2d, centers)
